# baseline (speedup 1.0000x reference)
import functools

import jax
import jax.numpy as jnp
from jax import lax
from jax.experimental import pallas as pl
from jax.experimental.pallas import tpu as pltpu
from jax.experimental.pallas import tpu_sc as plsc

VOCAB = 100000
D = 128
BATCH = 4096
SEQ = 200
N = BATCH * SEQ
NC, NS = 2, 16
NW = NC * NS
PER_W = N // NW
G = 128
STEPS = PER_W // G
NBUF = 4
BODIES = STEPS // NBUF
IDXROWS = PER_W // G


def _sc_body(seq_hbm, lab_hbm, tok_hbm, seg_hbm, out_hbm,
             idx_v, lab_v, rows0, rows1, rows2, rows3,
             gsem0, gsem1, gsem2, gsem3, ssem0, ssem1, ssem2, ssem3):
    wid = lax.axis_index("s") * NC + lax.axis_index("c")
    base = wid * PER_W
    rows = (rows0, rows1, rows2, rows3)
    gsems = (gsem0, gsem1, gsem2, gsem3)
    ssems = (ssem0, ssem1, ssem2, ssem3)

    pltpu.sync_copy(seq_hbm.at[pl.ds(pl.multiple_of(base // G, 8), IDXROWS)],
                    idx_v)
    pltpu.sync_copy(lab_hbm.at[pl.ds(pl.multiple_of(base // G, 8), IDXROWS)],
                    lab_v)

    def loop_body(i, _):
        s0i = i * NBUF
        gcp = [pltpu.async_copy(tok_hbm.at[idx_v.at[s0i + b]], rows[b],
                                gsems[b])
               for b in range(NBUF)]
        scp = []
        for b in range(NBUF):
            gcp[b].wait()
            pltpu.async_copy(seg_hbm.at[lab_v.at[s0i + b]], rows[b],
                             gsems[b], add=True).wait()
            off = base + (s0i + b) * G
            scp.append(pltpu.async_copy(
                rows[b], out_hbm.at[pl.ds(pl.multiple_of(off, G), G)],
                ssems[b]))
        for cp in scp:
            cp.wait()
        return 0

    lax.fori_loop(0, BODIES, loop_body, 0)


@jax.jit
def _embed(seq2d, lab2d, token_table, segment_table):
    fn = functools.partial(
        pl.kernel,
        out_type=jax.ShapeDtypeStruct((N, D), jnp.float32),
        mesh=plsc.VectorSubcoreMesh(core_axis_name="c", subcore_axis_name="s"),
        scratch_types=[
            pltpu.VMEM((IDXROWS, G), jnp.int32),
            pltpu.VMEM((IDXROWS, G), jnp.int32),
            pltpu.VMEM((G, D), jnp.float32),
            pltpu.VMEM((G, D), jnp.float32),
            pltpu.VMEM((G, D), jnp.float32),
            pltpu.VMEM((G, D), jnp.float32),
            pltpu.SemaphoreType.DMA,
            pltpu.SemaphoreType.DMA,
            pltpu.SemaphoreType.DMA,
            pltpu.SemaphoreType.DMA,
            pltpu.SemaphoreType.DMA,
            pltpu.SemaphoreType.DMA,
            pltpu.SemaphoreType.DMA,
            pltpu.SemaphoreType.DMA,
        ],
    )(_sc_body)
    return fn(seq2d, lab2d, token_table, segment_table)


def kernel(sequence, segment_label, token_table, segment_table):
    seq2d = sequence.reshape(N // G, G)
    lab2d = segment_label.reshape(N // G, G)
    out = _embed(seq2d, lab2d, token_table, segment_table)
    return out.reshape(BATCH, SEQ, D)


# speedup vs baseline: 21.4316x; 21.4316x over previous
import functools

import jax
import jax.numpy as jnp
from jax import lax
from jax.experimental import pallas as pl
from jax.experimental.pallas import tpu as pltpu
from jax.experimental.pallas import tpu_sc as plsc

VOCAB = 100000
D = 128
BATCH = 4096
SEQ = 200
N = BATCH * SEQ
NC, NS = 2, 16
NW = NC * NS
PER_W = N // NW
G = 128
STEPS = PER_W // G
SUBS = 8
NBUF = 7
BODIES = STEPS // SUBS
LANES = 16
DB = D // LANES


def _sc_body(seq_hbm, lab_hbm, tok_hbm, seg_hbm, out_hbm,
             idx_v, lab_v, seg_v,
             rows0, rows1, rows2, rows3, rows4, rows5, rows6,
             gsem0, gsem1, gsem2, gsem3, gsem4, gsem5, gsem6, gsem7,
             ssem0, ssem1, ssem2, ssem3, ssem4, ssem5, ssem6, ssem7):
    wid = lax.axis_index("s") * NC + lax.axis_index("c")
    base = wid * PER_W
    rows = (rows0, rows1, rows2, rows3, rows4, rows5, rows6)
    gsems = (gsem0, gsem1, gsem2, gsem3, gsem4, gsem5, gsem6, gsem7)
    ssems = (ssem0, ssem1, ssem2, ssem3, ssem4, ssem5, ssem6, ssem7)

    pltpu.sync_copy(seg_hbm, seg_v)
    s0 = [seg_v[0, pl.ds(db * LANES, LANES)] for db in range(DB)]
    d1 = [seg_v[1, pl.ds(db * LANES, LANES)] - s0[db] for db in range(DB)]
    d2 = [seg_v[2, pl.ds(db * LANES, LANES)] - seg_v[1, pl.ds(db * LANES, LANES)]
          for db in range(DB)]

    def compute(k, b):
        def grp_body(g, _):
            lab16 = lab_v[k, pl.ds(g * LANES, LANES)]
            for i in range(LANES):
                labi = lab16[i]
                a1 = jnp.full((LANES,),
                              jnp.minimum(labi, 1), jnp.int32).astype(jnp.float32)
                a2 = jnp.full((LANES,),
                              jnp.maximum(labi - 1, 0), jnp.int32).astype(jnp.float32)
                t = g * LANES + i
                for db in range(DB):
                    sl = pl.ds(db * LANES, LANES)
                    sv = s0[db] + a1 * d1[db] + a2 * d2[db]
                    rows[b][t, sl] = rows[b][t, sl] + sv
            return 0
        lax.fori_loop(0, G // LANES, grp_body, 0)

    def loop_body(i, _):
        srow = pl.multiple_of(base // G + i * SUBS, 8)
        pltpu.sync_copy(seq_hbm.at[pl.ds(srow, SUBS)], idx_v)
        pltpu.sync_copy(lab_hbm.at[pl.ds(srow, SUBS)], lab_v)
        gcp = [pltpu.async_copy(tok_hbm.at[idx_v.at[k]], rows[k], gsems[k])
               for k in range(NBUF)]
        scp = []
        for k in range(SUBS):
            b = k % NBUF
            if k == 4:
                scp[0].wait()
                gcp.append(pltpu.async_copy(tok_hbm.at[idx_v.at[NBUF]],
                                            rows[0], gsems[NBUF]))
            gcp[k].wait()
            compute(k, b)
            off = base + (i * SUBS + k) * G
            scp.append(pltpu.async_copy(
                rows[b], out_hbm.at[pl.ds(pl.multiple_of(off, G), G)],
                ssems[k]))
        for cp in scp[1:]:
            cp.wait()
        return 0

    lax.fori_loop(0, BODIES, loop_body, 0)


@jax.jit
def _embed(seq2d, lab2d, token_table, segment_table):
    fn = functools.partial(
        pl.kernel,
        out_type=jax.ShapeDtypeStruct((N, D), jnp.float32),
        mesh=plsc.VectorSubcoreMesh(core_axis_name="c", subcore_axis_name="s"),
        scratch_types=(
            [pltpu.VMEM((SUBS, G), jnp.int32),
             pltpu.VMEM((SUBS, G), jnp.int32),
             pltpu.VMEM((3, D), jnp.float32)]
            + [pltpu.VMEM((G, D), jnp.float32)] * NBUF
            + [pltpu.SemaphoreType.DMA] * (2 * SUBS)
        ),
    )(_sc_body)
    return fn(seq2d, lab2d, token_table, segment_table)


def kernel(sequence, segment_label, token_table, segment_table):
    seq2d = sequence.reshape(N // G, G)
    lab2d = segment_label.reshape(N // G, G)
    out = _embed(seq2d, lab2d, token_table, segment_table)
    return out.reshape(BATCH, SEQ, D)


# speedup vs baseline: 21.9871x; 1.0259x over previous
import functools

import jax
import jax.numpy as jnp
from jax import lax
from jax.experimental import pallas as pl
from jax.experimental.pallas import tpu as pltpu
from jax.experimental.pallas import tpu_sc as plsc

VOCAB = 100000
D = 128
BATCH = 4096
SEQ = 200
N = BATCH * SEQ
NSEG = 3
NC, NS = 2, 16
NW = NC * NS
PER_W = N // NW
G = 128
STEPS = PER_W // G
SUBS = 8
NBUF = 7
BODIES = STEPS // SUBS
LANES = 16
VBLK = 2000
VGRID = VOCAB // VBLK


def _build_body(tok_ref, seg_ref, out_ref):
    l = pl.program_id(0) // VGRID
    out_ref[...] = tok_ref[...] + seg_ref[pl.ds(l, 1), :]


@jax.jit
def _build_fused(token_table, seg8):
    return pl.pallas_call(
        _build_body,
        grid=(NSEG * VGRID,),
        in_specs=[
            pl.BlockSpec((VBLK, D), lambda i: (i % VGRID, 0)),
            pl.BlockSpec((8, D), lambda i: (0, 0)),
        ],
        out_specs=pl.BlockSpec((VBLK, D), lambda i: (i, 0)),
        out_shape=jax.ShapeDtypeStruct((NSEG * VOCAB, D), jnp.float32),
    )(token_table, seg8)


def _sc_body(seq_hbm, lab_hbm, fused_hbm, out_hbm,
             idx_v, lab_v,
             rows0, rows1, rows2, rows3, rows4, rows5, rows6,
             gsem0, gsem1, gsem2, gsem3, gsem4, gsem5, gsem6, gsem7,
             ssem0, ssem1, ssem2, ssem3, ssem4, ssem5, ssem6, ssem7):
    wid = lax.axis_index("s") * NC + lax.axis_index("c")
    base = wid * PER_W
    rows = (rows0, rows1, rows2, rows3, rows4, rows5, rows6)
    gsems = (gsem0, gsem1, gsem2, gsem3, gsem4, gsem5, gsem6, gsem7)
    ssems = (ssem0, ssem1, ssem2, ssem3, ssem4, ssem5, ssem6, ssem7)

    def loop_body(i, _):
        srow = pl.multiple_of(base // G + i * SUBS, 8)
        pltpu.sync_copy(seq_hbm.at[pl.ds(srow, SUBS)], idx_v)
        pltpu.sync_copy(lab_hbm.at[pl.ds(srow, SUBS)], lab_v)
        for k in range(SUBS):
            for g in range(G // LANES):
                sl = pl.ds(g * LANES, LANES)
                idx_v[k, sl] = idx_v[k, sl] + lab_v[k, sl] * VOCAB
        gcp = [pltpu.async_copy(fused_hbm.at[idx_v.at[k]], rows[k], gsems[k])
               for k in range(NBUF)]
        scp = []
        for k in range(SUBS):
            b = k % NBUF
            if k == 4:
                scp[0].wait()
                gcp.append(pltpu.async_copy(fused_hbm.at[idx_v.at[NBUF]],
                                            rows[0], gsems[NBUF]))
            gcp[k].wait()
            off = base + (i * SUBS + k) * G
            scp.append(pltpu.async_copy(
                rows[b], out_hbm.at[pl.ds(pl.multiple_of(off, G), G)],
                ssems[k]))
        for cp in scp[1:]:
            cp.wait()
        return 0

    lax.fori_loop(0, BODIES, loop_body, 0)


@jax.jit
def _embed(seq2d, lab2d, fused):
    fn = functools.partial(
        pl.kernel,
        out_type=jax.ShapeDtypeStruct((N, D), jnp.float32),
        mesh=plsc.VectorSubcoreMesh(core_axis_name="c", subcore_axis_name="s"),
        scratch_types=(
            [pltpu.VMEM((SUBS, G), jnp.int32),
             pltpu.VMEM((SUBS, G), jnp.int32)]
            + [pltpu.VMEM((G, D), jnp.float32)] * NBUF
            + [pltpu.SemaphoreType.DMA] * (2 * SUBS)
        ),
    )(_sc_body)
    return fn(seq2d, lab2d, fused)


def kernel(sequence, segment_label, token_table, segment_table):
    seq2d = sequence.reshape(N // G, G)
    lab2d = segment_label.reshape(N // G, G)
    seg8 = jnp.pad(segment_table, ((0, 8 - NSEG), (0, 0)))
    fused = _build_fused(token_table, seg8)
    out = _embed(seq2d, lab2d, fused)
    return out.reshape(BATCH, SEQ, D)


# speedup vs baseline: 24.4505x; 1.1120x over previous
import functools

import jax
import jax.numpy as jnp
from jax import lax
from jax.experimental import pallas as pl
from jax.experimental.pallas import tpu as pltpu
from jax.experimental.pallas import tpu_sc as plsc

VOCAB = 100000
D = 128
BATCH = 4096
SEQ = 200
N = BATCH * SEQ
NSEG = 3
NC, NS = 2, 16
NW = NC * NS
PER_W = N // NW
G = 128
STEPS = PER_W // G
SUBS = 8
NBUF = 7
BODIES = STEPS // SUBS
LANES = 16
VBLK = 4000
VGRID = VOCAB // VBLK


def _build_body(tok_ref, seg_ref, out_ref):
    l = pl.program_id(0) % NSEG
    out_ref[...] = tok_ref[...] + seg_ref[pl.ds(l, 1), :]


@jax.jit
def _build_fused(token_table, seg8):
    return pl.pallas_call(
        _build_body,
        grid=(NSEG * VGRID,),
        in_specs=[
            pl.BlockSpec((VBLK, D), lambda i: (i // NSEG, 0)),
            pl.BlockSpec((8, D), lambda i: (0, 0)),
        ],
        out_specs=pl.BlockSpec((VBLK, D),
                               lambda i: ((i % NSEG) * VGRID + i // NSEG, 0)),
        out_shape=jax.ShapeDtypeStruct((NSEG * VOCAB, D), jnp.float32),
    )(token_table, seg8)


def _sc_body(seq_hbm, lab_hbm, fused_hbm, out_hbm,
             idx_v, lab_v,
             rows0, rows1, rows2, rows3, rows4, rows5, rows6,
             gsem0, gsem1, gsem2, gsem3, gsem4, gsem5, gsem6, gsem7,
             ssem0, ssem1, ssem2, ssem3, ssem4, ssem5, ssem6, ssem7):
    wid = lax.axis_index("s") * NC + lax.axis_index("c")
    base = wid * PER_W
    rows = (rows0, rows1, rows2, rows3, rows4, rows5, rows6)
    gsems = (gsem0, gsem1, gsem2, gsem3, gsem4, gsem5, gsem6, gsem7)
    ssems = (ssem0, ssem1, ssem2, ssem3, ssem4, ssem5, ssem6, ssem7)

    def loop_body(i, _):
        srow = pl.multiple_of(base // G + i * SUBS, 8)
        pltpu.sync_copy(seq_hbm.at[pl.ds(srow, SUBS)], idx_v)
        pltpu.sync_copy(lab_hbm.at[pl.ds(srow, SUBS)], lab_v)
        for k in range(SUBS):
            for g in range(G // LANES):
                sl = pl.ds(g * LANES, LANES)
                idx_v[k, sl] = idx_v[k, sl] + lab_v[k, sl] * VOCAB
        gcp = [pltpu.async_copy(fused_hbm.at[idx_v.at[k]], rows[k], gsems[k])
               for k in range(NBUF)]
        scp = []
        for k in range(SUBS):
            b = k % NBUF
            if k == 4:
                scp[0].wait()
                gcp.append(pltpu.async_copy(fused_hbm.at[idx_v.at[NBUF]],
                                            rows[0], gsems[NBUF]))
            gcp[k].wait()
            off = base + (i * SUBS + k) * G
            scp.append(pltpu.async_copy(
                rows[b], out_hbm.at[pl.ds(pl.multiple_of(off, G), G)],
                ssems[k]))
        for cp in scp[1:]:
            cp.wait()
        return 0

    lax.fori_loop(0, BODIES, loop_body, 0)


@jax.jit
def _embed(seq2d, lab2d, fused):
    fn = functools.partial(
        pl.kernel,
        out_type=jax.ShapeDtypeStruct((N, D), jnp.float32),
        mesh=plsc.VectorSubcoreMesh(core_axis_name="c", subcore_axis_name="s"),
        scratch_types=(
            [pltpu.VMEM((SUBS, G), jnp.int32),
             pltpu.VMEM((SUBS, G), jnp.int32)]
            + [pltpu.VMEM((G, D), jnp.float32)] * NBUF
            + [pltpu.SemaphoreType.DMA] * (2 * SUBS)
        ),
    )(_sc_body)
    return fn(seq2d, lab2d, fused)


def kernel(sequence, segment_label, token_table, segment_table):
    seq2d = sequence.reshape(N // G, G)
    lab2d = segment_label.reshape(N // G, G)
    seg8 = jnp.pad(segment_table, ((0, 8 - NSEG), (0, 0)))
    fused = _build_fused(token_table, seg8)
    out = _embed(seq2d, lab2d, fused)
    return out.reshape(BATCH, SEQ, D)
